# Initial kernel scaffold; baseline (speedup 1.0000x reference)
import jax
import jax.numpy as jnp
from jax.experimental import pallas as pl


def kernel(lengths, table):
    raise NotImplementedError("write your pallas kernel here")


# speedup vs baseline: 1.6112x; 1.6112x over previous
import numpy as np

import jax
import jax.numpy as jnp
from jax import lax
from jax.experimental import pallas as pl
from jax.experimental.pallas import tpu as pltpu
from jax.experimental.pallas import tpu_sc as plsc

_BINS = (1, 2, 3, 4, 8, 16, 32, 64, 128, 256, 384)

_B = 16384
_D = 20
_NC, _NS, _L = 2, 16, 16
_NW = _NC * _NS
_BPW = _B // _NW
_OPW = _BPW * _D
_BLK = 80
_NBLK = _OPW // _BLK

_S = np.arange(_BLK).reshape(5, _L)
_QC = (_S // _D).astype(np.int32)
_RC = (_S % _D).astype(np.int32)


def _body(len_hbm, tab_hbm, out_hbm, len_v, gofs_v, tab_v, out_v):
    wid = lax.axis_index("s") * _NC + lax.axis_index("c")
    base = wid * _BPW
    pltpu.sync_copy(len_hbm.at[pl.ds(base, _BPW)], len_v)
    pltpu.sync_copy(tab_hbm, tab_v)
    for j in range(_BPW // _L):
        v = len_v[pl.ds(j * _L, _L)]
        acc = lax.shift_right_logical(_BINS[0] - v, 31)
        for b in _BINS[1:]:
            acc = acc + lax.shift_right_logical(b - v, 31)
        gofs_v[pl.ds(j * _L, _L)] = acc * _D
    lane = lax.iota(jnp.int32, _L)
    qc, rc = [], []
    for t in range(5):
        s = lane + t * _L
        q = lax.shift_right_logical(s * 52429, 20)
        qc.append(q)
        rc.append(s - q * _D)

    def blk(m, carry):
        for t in range(5):
            q = qc[t] + m * 4
            g = plsc.load_gather(gofs_v, [q]) + rc[t]
            out_v[pl.ds(m * _BLK + t * _L, _L)] = plsc.load_gather(tab_v, [g])
        return carry

    lax.fori_loop(0, _NBLK, blk, 0)
    pltpu.sync_copy(out_v, out_hbm.at[pl.ds(wid * _OPW, _OPW)])


def kernel(lengths, table):
    mesh = plsc.VectorSubcoreMesh(core_axis_name="c", subcore_axis_name="s")
    out_flat = pl.kernel(
        _body,
        out_type=jax.ShapeDtypeStruct((_B * _D,), jnp.float32),
        mesh=mesh,
        scratch_types=[
            pltpu.VMEM((_BPW,), jnp.int32),
            pltpu.VMEM((_BPW,), jnp.int32),
            pltpu.VMEM((table.size,), jnp.float32),
            pltpu.VMEM((_OPW,), jnp.float32),
        ],
        compiler_params=pltpu.CompilerParams(needs_layout_passes=False),
    )(lengths, table.reshape(-1))
    return out_flat.reshape(_B, _D)


# speedup vs baseline: 2.1220x; 1.3170x over previous
import jax
import jax.numpy as jnp
from jax import lax
from jax.experimental import pallas as pl
from jax.experimental.pallas import tpu as pltpu
from jax.experimental.pallas import tpu_sc as plsc

_BINS = (1, 2, 3, 4, 8, 16, 32, 64, 128, 256, 384)

_B = 16384
_D = 20
_NC, _NS, _L = 2, 16, 16
_NW = _NC * _NS
_BPW = _B // _NW
_RPI = 16


def _body(len_hbm, tab_hbm, out_hbm, len_v, gofs_v, tab_v, out_v):
    wid = lax.axis_index("s") * _NC + lax.axis_index("c")
    base = wid * _BPW
    pltpu.sync_copy(len_hbm.at[pl.ds(base, _BPW)], len_v)
    pltpu.sync_copy(tab_hbm, tab_v)
    for j in range(_BPW // _L):
        v = len_v[pl.ds(j * _L, _L)]
        acc = lax.shift_right_logical(_BINS[0] - v, 31)
        for b in _BINS[1:]:
            acc = acc + lax.shift_right_logical(b - v, 31)
        gofs_v[pl.ds(j * _L, _L)] = acc * _D
    lane = lax.iota(jnp.int32, _L)

    def blk(i, carry):
        r0 = i * _RPI
        vrow = gofs_v[pl.ds(r0, _L)]
        for r in range(_RPI):
            g = vrow[r] + lane
            out_v[r0 + r, pl.ds(0, _L)] = plsc.load_gather(tab_v, [g])
            out_v[r0 + r, pl.ds(_D - _L, _L)] = plsc.load_gather(
                tab_v, [g + (_D - _L)]
            )
        return carry

    lax.fori_loop(0, _BPW // _RPI, blk, 0)
    pltpu.sync_copy(out_v, out_hbm.at[pl.ds(base, _BPW)])


def kernel(lengths, table):
    mesh = plsc.VectorSubcoreMesh(core_axis_name="c", subcore_axis_name="s")
    return pl.kernel(
        _body,
        out_type=jax.ShapeDtypeStruct((_B, _D), jnp.float32),
        mesh=mesh,
        scratch_types=[
            pltpu.VMEM((_BPW,), jnp.int32),
            pltpu.VMEM((_BPW,), jnp.int32),
            pltpu.VMEM((_D * 12,), jnp.float32),
            pltpu.VMEM((_BPW, _D), jnp.float32),
        ],
        compiler_params=pltpu.CompilerParams(needs_layout_passes=False),
    )(lengths, table.reshape(-1))
